# baseline (speedup 1.0000x reference)
import functools

import jax
import jax.numpy as jnp
from jax import lax
from jax.experimental import pallas as pl
from jax.experimental.pallas import tpu as pltpu
from jax.experimental.pallas import tpu_sc as plsc

_NB = 4
_RB = 128


def _emb_call(num_cores, num_subcores, R, S, D):
    mesh = plsc.VectorSubcoreMesh(core_axis_name="c", subcore_axis_name="s")
    n_workers = num_cores * num_subcores
    GD = D // 8
    RT = R // _RB
    assert RT == n_workers and S % _NB == 0 and S >= 2 * _NB

    @functools.partial(
        pl.kernel,
        mesh=mesh,
        out_type=jax.ShapeDtypeStruct((S, GD, RT, 8, _RB), jnp.float32),
        compiler_params=pltpu.CompilerParams(
            use_tc_tiling_on_sc=False, needs_layout_passes=False
        ),
        scratch_types=[
            pltpu.VMEM((S, _RB), jnp.int32),
            pltpu.VMEM((_NB, _RB, D), jnp.float32),
            pltpu.VMEM((_NB, GD, 8, _RB), jnp.float32),
            pltpu.SemaphoreType.DMA((_NB,)),
            pltpu.SemaphoreType.DMA((_NB,)),
        ],
    )
    def emb(xt_hbm, table_hbm, out_hbm, idx_v, rows_v, tbuf_v, gsems, ssems):
        wid = lax.axis_index("s") * num_cores + lax.axis_index("c")
        pltpu.sync_copy(xt_hbm.at[:, pl.ds(wid * _RB, _RB)], idx_v)
        lanes = jax.lax.iota(jnp.int32, 16)
        rids = [lanes + 16 * q for q in range(_RB // 16)]

        def gather(c, b):
            pltpu.async_copy(
                table_hbm.at[idx_v.at[c]], rows_v.at[b], gsems.at[b]
            )

        def transpose(b):
            rows = rows_v.at[b]
            tb = tbuf_v.at[b]

            def per_d(d, carry):
                g = d // 8
                gi = d % 8
                col = jnp.broadcast_to(d, (16,)).astype(jnp.int32)
                for q in range(_RB // 16):
                    v = plsc.load_gather(rows, [rids[q], col])
                    tb[g, gi, pl.ds(16 * q, 16)] = v
                return carry

            lax.fori_loop(0, D, per_d, 0)

        def scatter(c, b):
            pltpu.async_copy(
                tbuf_v.at[b], out_hbm.at[c, :, wid], ssems.at[b]
            )

        def drain_g(b):
            pltpu.make_async_copy(
                table_hbm.at[pl.ds(0, _RB)], rows_v.at[b], gsems.at[b]
            ).wait()

        def drain_s(b):
            pltpu.make_async_copy(
                tbuf_v.at[b], out_hbm.at[0, :, 0], ssems.at[b]
            ).wait()

        def step(c, b, drain_scatter, prefetch):
            drain_g(b)
            if drain_scatter:
                drain_s(b)
            transpose(b)
            scatter(c, b)
            if prefetch:
                gather(c + _NB, b)

        for c in range(_NB):
            gather(c, c)
        for b in range(_NB):
            step(b, b, drain_scatter=False, prefetch=True)

        def body(i, carry):
            c0 = i * _NB
            for b in range(_NB):
                step(c0 + b, b, drain_scatter=True, prefetch=True)
            return carry

        lax.fori_loop(1, S // _NB - 1, body, 0)

        c0 = S - _NB
        for b in range(_NB):
            step(c0 + b, b, drain_scatter=True, prefetch=False)
        for b in range(_NB):
            drain_s(b)

    return emb


def kernel(x, table):
    R, S = x.shape
    V, D = table.shape
    info = plsc.get_sparse_core_info()
    xt = x.T.astype(jnp.int32)
    out5 = _emb_call(info.num_cores, info.num_subcores, R, S, D)(xt, table)
    return out5.transpose(2, 4, 0, 1, 3).reshape(R, S, D)


# speedup vs baseline: 2.6195x; 2.6195x over previous
import functools

import jax
import jax.numpy as jnp
from jax import lax
from jax.experimental import pallas as pl
from jax.experimental.pallas import tpu as pltpu
from jax.experimental.pallas import tpu_sc as plsc

_NB = 4
_RB = 128


def _emb_call(num_cores, num_subcores, R, S, D):
    mesh = plsc.VectorSubcoreMesh(core_axis_name="c", subcore_axis_name="s")
    n_workers = num_cores * num_subcores
    GD = D // 8
    RT = R // _RB
    assert RT == n_workers and S % _NB == 0 and S >= 2 * _NB

    @functools.partial(
        pl.kernel,
        mesh=mesh,
        out_type=jax.ShapeDtypeStruct((S, GD, RT, 8, _RB), jnp.float32),
        compiler_params=pltpu.CompilerParams(
            use_tc_tiling_on_sc=False, needs_layout_passes=False
        ),
        scratch_types=[
            pltpu.VMEM((S, _RB), jnp.int32),
            pltpu.VMEM((_NB, _RB, D), jnp.float32),
            pltpu.VMEM((_NB, GD, 8, _RB), jnp.float32),
            pltpu.SemaphoreType.DMA((_NB,)),
            pltpu.SemaphoreType.DMA((_NB,)),
        ],
    )
    def emb(xt_hbm, table_hbm, out_hbm, idx_v, rows_v, tbuf_v, gsems, ssems):
        wid = lax.axis_index("s") * num_cores + lax.axis_index("c")
        pltpu.sync_copy(xt_hbm.at[:, pl.ds(wid * _RB, _RB)], idx_v)
        lanes = jax.lax.iota(jnp.int32, 16)
        rids = [lanes + 16 * q for q in range(_RB // 16)]

        def gather(c, b):
            pltpu.async_copy(
                table_hbm.at[idx_v.at[c]], rows_v.at[b], gsems.at[b]
            )

        def transpose(b):
            rows = rows_v.at[b]
            tb = tbuf_v.at[b]

            def per_d0(d0, carry):
                dcol = (d0 + lanes) & (D - 1)
                g = jnp.right_shift(dcol, 3)
                gi = dcol & 7
                for q in range(_RB // 16):
                    v = plsc.load_gather(rows, [rids[q], dcol])
                    plsc.store_scatter(tb, [g, gi, rids[q]], v)
                return carry

            lax.fori_loop(0, D, per_d0, 0)

        def scatter(c, b):
            pltpu.async_copy(
                tbuf_v.at[b], out_hbm.at[c, :, wid], ssems.at[b]
            )

        def drain_g(b):
            pltpu.make_async_copy(
                table_hbm.at[pl.ds(0, _RB)], rows_v.at[b], gsems.at[b]
            ).wait()

        def drain_s(b):
            pltpu.make_async_copy(
                tbuf_v.at[b], out_hbm.at[0, :, 0], ssems.at[b]
            ).wait()

        def step(c, b, drain_scatter, prefetch):
            drain_g(b)
            if drain_scatter:
                drain_s(b)
            transpose(b)
            scatter(c, b)
            if prefetch:
                gather(c + _NB, b)

        for c in range(_NB):
            gather(c, c)
        for b in range(_NB):
            step(b, b, drain_scatter=False, prefetch=True)

        def body(i, carry):
            c0 = i * _NB
            for b in range(_NB):
                step(c0 + b, b, drain_scatter=True, prefetch=True)
            return carry

        lax.fori_loop(1, S // _NB - 1, body, 0)

        c0 = S - _NB
        for b in range(_NB):
            step(c0 + b, b, drain_scatter=True, prefetch=False)
        for b in range(_NB):
            drain_s(b)

    return emb


def kernel(x, table):
    R, S = x.shape
    V, D = table.shape
    info = plsc.get_sparse_core_info()
    xt = x.T.astype(jnp.int32)
    out5 = _emb_call(info.num_cores, info.num_subcores, R, S, D)(xt, table)
    return out5.transpose(2, 4, 0, 1, 3).reshape(R, S, D)


# speedup vs baseline: 5.8336x; 2.2270x over previous
import functools

import jax
import jax.numpy as jnp
from jax import lax
from jax.experimental import pallas as pl
from jax.experimental.pallas import tpu as pltpu
from jax.experimental.pallas import tpu_sc as plsc

_NB = 4
_RB = 128


def _emb_call(num_cores, num_subcores, R, S, D):
    mesh = plsc.VectorSubcoreMesh(core_axis_name="c", subcore_axis_name="s")
    n_workers = num_cores * num_subcores
    GD = D // 8
    RT = R // _RB
    assert RT == n_workers and S % _NB == 0 and S >= 2 * _NB

    @functools.partial(
        pl.kernel,
        mesh=mesh,
        out_type=jax.ShapeDtypeStruct((S, GD, RT, 8, _RB), jnp.float32),
        compiler_params=pltpu.CompilerParams(
            use_tc_tiling_on_sc=False, needs_layout_passes=False
        ),
        scratch_types=[
            pltpu.VMEM((S, _RB), jnp.int32),
            pltpu.VMEM((_NB, _RB, D), jnp.float32),
            pltpu.VMEM((_NB, GD, 8, _RB), jnp.float32),
            pltpu.SemaphoreType.DMA((_NB,)),
            pltpu.SemaphoreType.DMA((_NB,)),
        ],
    )
    def emb(xt_hbm, table_hbm, out_hbm, idx_v, rows_v, tbuf_v, gsems, ssems):
        wid = lax.axis_index("s") * num_cores + lax.axis_index("c")
        pltpu.sync_copy(xt_hbm.at[:, pl.ds(wid * _RB, _RB)], idx_v)
        lanes = jax.lax.iota(jnp.int32, 16)
        rids = [lanes + 16 * q for q in range(_RB // 16)]

        def gather(c, b):
            pltpu.async_copy(
                table_hbm.at[idx_v.at[c]], rows_v.at[b], gsems.at[b]
            )

        def transpose(b):
            rows = rows_v.at[b]
            tb = tbuf_v.at[b]

            @plsc.parallel_loop(0, D, unroll=4)
            def per_d0(d0):
                dcol = (d0 + lanes) & (D - 1)
                g = jnp.right_shift(dcol, 3)
                gi = dcol & 7
                for q in range(_RB // 16):
                    v = plsc.load_gather(rows, [rids[q], dcol])
                    plsc.store_scatter(tb, [g, gi, rids[q]], v)

        def scatter(c, b):
            pltpu.async_copy(
                tbuf_v.at[b], out_hbm.at[c, :, wid], ssems.at[b]
            )

        def drain_g(b):
            pltpu.make_async_copy(
                table_hbm.at[pl.ds(0, _RB)], rows_v.at[b], gsems.at[b]
            ).wait()

        def drain_s(b):
            pltpu.make_async_copy(
                tbuf_v.at[b], out_hbm.at[0, :, 0], ssems.at[b]
            ).wait()

        def step(c, b, drain_scatter, prefetch):
            drain_g(b)
            if drain_scatter:
                drain_s(b)
            transpose(b)
            scatter(c, b)
            if prefetch:
                gather(c + _NB, b)

        for c in range(_NB):
            gather(c, c)
        for b in range(_NB):
            step(b, b, drain_scatter=False, prefetch=True)

        def body(i, carry):
            c0 = i * _NB
            for b in range(_NB):
                step(c0 + b, b, drain_scatter=True, prefetch=True)
            return carry

        lax.fori_loop(1, S // _NB - 1, body, 0)

        c0 = S - _NB
        for b in range(_NB):
            step(c0 + b, b, drain_scatter=True, prefetch=False)
        for b in range(_NB):
            drain_s(b)

    return emb


def kernel(x, table):
    R, S = x.shape
    V, D = table.shape
    info = plsc.get_sparse_core_info()
    xt = x.T.astype(jnp.int32)
    out5 = _emb_call(info.num_cores, info.num_subcores, R, S, D)(xt, table)
    return out5.transpose(2, 4, 0, 1, 3).reshape(R, S, D)
